# baseline (speedup 1.0000x reference)
import functools

import jax
import jax.numpy as jnp
from jax import lax
from jax.experimental import pallas as pl
from jax.experimental.pallas import tpu as pltpu
from jax.experimental.pallas import tpu_sc as plsc

_IGNORE = 255
_RATIO = 0.1
_THR = 0.5
_SHAPE = (16, 512, 512)
_N = 16 * 512 * 512

_NC = 2
_NS = 16
_L = 16
_NW = _NC * _NS
_NBAND = 1024
_BPW = _NBAND // _NW
_BVEC = 8 * 512 // _L
_NSTAT = 5
_B = 1024

_SC_BATCHES = 12
_TC_BATCHES = 16 - _SC_BATCHES
_BPW_SC = _SC_BATCHES * 64 // _NW
_CPW = _BPW_SC // 2
_CVEC = 16 * 512 // _L


def _mesh():
    return plsc.VectorSubcoreMesh(
        core_axis_name="c", subcore_axis_name="s", num_cores=_NC, num_subcores=_NS
    )


@functools.lru_cache(maxsize=None)
def _build_stats_sc():
    return pl.kernel(
        _stats_body,
        mesh=_mesh(),
        out_type=jax.ShapeDtypeStruct((_NW, _NSTAT * _L), jnp.float32),
        compiler_params=pltpu.CompilerParams(use_tc_tiling_on_sc=True),
        scratch_types=[
            pltpu.VMEM((2, 8, 512), jnp.float32),
            pltpu.VMEM((2, 8, 512), jnp.int32),
            pltpu.VMEM((_NSTAT * _L,), jnp.float32),
            pltpu.SemaphoreType.DMA,
            pltpu.SemaphoreType.DMA,
        ],
    )


def _stats_body(loss_hbm, so_hbm, out_hbm, lbuf, obuf, stage, sem0, sem1):
    wid = lax.axis_index("s") * _NC + lax.axis_index("c")
    zeros = jnp.zeros((_L,), jnp.float32)
    sems = (sem0, sem1)

    def chunk_slices(u):
        band = wid * _BPW_SC + u
        b = band // 64
        r0 = (band % 64) * 8
        return b, r0

    def start(u, slot):
        b, r0 = chunk_slices(u)
        pltpu.async_copy(loss_hbm.at[b, pl.ds(r0, 8), :], lbuf.at[slot], sems[slot])
        pltpu.async_copy(so_hbm.at[b, pl.ds(r0, 8), :], obuf.at[slot], sems[slot])

    def wait(u, slot):
        b, r0 = chunk_slices(u)
        pltpu.make_async_copy(loss_hbm.at[b, pl.ds(r0, 8), :], lbuf.at[slot], sems[slot]).wait()
        pltpu.make_async_copy(so_hbm.at[b, pl.ds(r0, 8), :], obuf.at[slot], sems[slot]).wait()

    def compute(slot, accs):
        def vec_body(i, a):
            ns_cnt, h_cnt, h_sum, hs_cnt, hs_sum = a
            r = i // 32
            c = (i % 32) * _L
            ls = lbuf[slot, r, pl.ds(c, _L)]
            so = obuf[slot, r, pl.ds(c, _L)]
            one = jnp.ones((_L,), jnp.float32)
            zero = jnp.zeros((_L,), jnp.float32)
            so_f = so.astype(jnp.float32)
            hard = ls > _THR
            hard_f = jnp.where(hard, one, zero)
            hs_f = hard_f * so_f
            return (
                ns_cnt + so_f,
                h_cnt + hard_f,
                h_sum + jnp.where(hard, ls, zero),
                hs_cnt + hs_f,
                hs_sum + hs_f * ls,
            )

        return lax.fori_loop(0, _BVEC, vec_body, accs, unroll=8)

    start(0, 0)

    def loop_body(jj, accs):
        u0 = 2 * jj
        start(u0 + 1, 1)
        wait(u0, 0)
        accs = compute(0, accs)

        @pl.when(jj + 1 < _BPW_SC // 2)
        def _():
            start(u0 + 2, 0)

        wait(u0 + 1, 1)
        return compute(1, accs)

    accs = lax.fori_loop(0, _BPW_SC // 2, loop_body, (zeros,) * _NSTAT)
    for i in range(_NSTAT):
        stage[pl.ds(i * _L, _L)] = accs[i]
    pltpu.sync_copy(stage, out_hbm.at[wid])


@functools.lru_cache(maxsize=None)
def _build_valid_tc():
    return pl.pallas_call(
        _valid_tc_body,
        out_shape=jax.ShapeDtypeStruct((1, 1), jnp.float32),
        out_specs=pl.BlockSpec(memory_space=pltpu.SMEM),
    )


def _valid_tc_body(seg_ref, out_ref):
    valid = seg_ref[...] != _IGNORE
    out_ref[0, 0] = jnp.sum(jnp.where(valid, jnp.float32(1.0), jnp.float32(0.0)))


@functools.lru_cache(maxsize=None)
def _build_stats_tc():
    return pl.pallas_call(
        _stats_tc_body,
        grid=(_TC_BATCHES,),
        in_specs=[
            pl.BlockSpec((1, 512, 512), lambda i: (_SC_BATCHES + i, 0, 0)),
            pl.BlockSpec((1, 512, 512), lambda i: (_SC_BATCHES + i, 0, 0)),
        ],
        out_shape=tuple(
            jax.ShapeDtypeStruct((1, 1), jnp.float32) for _ in range(_NSTAT)
        ),
        out_specs=tuple(
            pl.BlockSpec(memory_space=pltpu.SMEM) for _ in range(_NSTAT)
        ),
    )


def _stats_tc_body(loss_ref, so_ref, ns_o, hc_o, hsum_o, hsc_o, hss_o):
    @pl.when(pl.program_id(0) == 0)
    def _():
        for o in (ns_o, hc_o, hsum_o, hsc_o, hss_o):
            o[0, 0] = jnp.float32(0.0)

    ls = loss_ref[0]
    so_f = so_ref[0].astype(jnp.float32)
    hard = ls > _THR
    hard_f = jnp.where(hard, jnp.float32(1.0), jnp.float32(0.0))
    hs_f = hard_f * so_f
    ns_o[0, 0] += jnp.sum(so_f)
    hc_o[0, 0] += jnp.sum(hard_f)
    hsum_o[0, 0] += jnp.sum(jnp.where(hard, ls, jnp.float32(0.0)))
    hsc_o[0, 0] += jnp.sum(hs_f)
    hss_o[0, 0] += jnp.sum(hs_f * ls)


@functools.lru_cache(maxsize=None)
def _build_hist_sc():
    return pl.kernel(
        _hist_body,
        mesh=_mesh(),
        out_type=(
            jax.ShapeDtypeStruct((_NW, 2 * _B), jnp.float32),
            jax.ShapeDtypeStruct((_NW, 2 * _B), jnp.float32),
        ),
        compiler_params=pltpu.CompilerParams(
            needs_layout_passes=False, use_tc_tiling_on_sc=True
        ),
        scratch_types=[
            pltpu.VMEM((8, 512), jnp.float32),
            pltpu.VMEM((8, 512), jnp.int32),
            pltpu.VMEM((2 * _B,), jnp.float32),
            pltpu.VMEM((2 * _B,), jnp.float32),
        ],
    )


def _hist_body(loss_hbm, so_hbm, cnt_hbm, sum_hbm, lbuf, obuf, hcnt, hsum):
    wid = lax.axis_index("s") * _NC + lax.axis_index("c")
    zeros = jnp.zeros((_L,), jnp.float32)
    ones = jnp.ones((_L,), jnp.float32)

    def zero_body(i, c):
        hcnt[pl.ds(i * _L, _L)] = zeros
        hsum[pl.ds(i * _L, _L)] = zeros
        return c

    lax.fori_loop(0, 2 * _B // _L, zero_body, 0)

    def band_body(t, carry):
        band = wid * _BPW + t
        b = band // 64
        r0 = (band % 64) * 8
        pltpu.sync_copy(loss_hbm.at[b, pl.ds(r0, 8), :], lbuf)
        pltpu.sync_copy(so_hbm.at[b, pl.ds(r0, 8), :], obuf)

        def vec_body(i, cc):
            r = i // 32
            c = (i % 32) * _L
            ls = lbuf[r, pl.ds(c, _L)]
            so = obuf[r, pl.ds(c, _L)]
            g = jnp.clip(so, 0, 1)
            bn = jnp.clip((ls * jnp.float32(_B)).astype(jnp.int32), 0, _B - 1)
            idx = bn + g * _B
            plsc.addupdate_scatter(hcnt, [idx], ones)
            plsc.addupdate_scatter(hsum, [idx], ls)
            return cc

        return lax.fori_loop(0, _BVEC, vec_body, carry)

    lax.fori_loop(0, _BPW, band_body, 0)
    pltpu.sync_copy(hcnt, cnt_hbm.at[wid])
    pltpu.sync_copy(hsum, sum_hbm.at[wid])


def _topk_sum_from_hist(cnt, sm, k):
    c = cnt[::-1]
    s = sm[::-1]
    cum = jnp.cumsum(c)
    take = jnp.clip(k.astype(jnp.float32) - (cum - c), 0.0, c)
    return jnp.sum(take * (s / jnp.maximum(c, 1.0)))


def kernel(loss, labels_seg, labels_so):
    parts = _build_stats_sc()(loss, labels_so)
    n_valid = _build_valid_tc()(labels_seg)[0, 0]
    tc_parts = _build_stats_tc()(loss, labels_so)
    p = jnp.sum(parts.reshape(_NW, _NSTAT, _L), axis=(0, 2))
    numel_s, h_cnt, h_sum, hs_cnt, hs_sum = (
        p[i] + tc_parts[i][0, 0] for i in range(_NSTAT)
    )

    n_min = (n_valid * jnp.float32(_RATIO)).astype(jnp.int32)
    cnt_s = hs_cnt.astype(jnp.int32)
    cnt_l = (h_cnt - hs_cnt).astype(jnp.int32)
    numel_s_i = numel_s.astype(jnp.int32)
    numel_l_i = jnp.int32(_N) - numel_s_i
    sum_s = hs_sum
    sum_l = h_sum - hs_sum

    k_s = jnp.where(cnt_s >= n_min, cnt_s, jnp.minimum(numel_s_i, n_min))
    k_l = jnp.where(cnt_l >= n_min, cnt_l, jnp.minimum(numel_l_i, n_min))
    need_s = cnt_s < n_min
    need_l = cnt_l < n_min

    def _fallback(_):
        hcnt, hsum = _build_hist_sc()(loss, labels_so)
        hc = jnp.sum(hcnt.reshape(_NW, 2, _B), axis=0)
        hs = jnp.sum(hsum.reshape(_NW, 2, _B), axis=0)
        return (
            _topk_sum_from_hist(hc[1], hs[1], k_s),
            _topk_sum_from_hist(hc[0], hs[0], k_l),
        )

    def _easy(_):
        return sum_s, sum_l

    fb_s, fb_l = lax.cond(need_s | need_l, _fallback, _easy, None)
    s_s = jnp.where(need_s, fb_s, sum_s)
    s_l = jnp.where(need_l, fb_l, sum_l)
    return (s_s + s_l) / (k_s + k_l).astype(jnp.float32)


# speedup vs baseline: 1.0451x; 1.0451x over previous
import functools

import jax
import jax.numpy as jnp
from jax import lax
from jax.experimental import pallas as pl
from jax.experimental.pallas import tpu as pltpu
from jax.experimental.pallas import tpu_sc as plsc

_IGNORE = 255
_RATIO = 0.1
_THR = 0.5
_SHAPE = (16, 512, 512)
_N = 16 * 512 * 512

_NC = 2
_NS = 16
_L = 16
_NW = _NC * _NS
_NBAND = 1024
_BPW = _NBAND // _NW
_BVEC = 8 * 512 // _L
_NSTAT = 5
_B = 1024

_SC_BATCHES = 9
_TC_BATCHES = 16 - _SC_BATCHES
_BPW_SC = _SC_BATCHES * 64 // _NW
_CPW = _BPW_SC // 2
_CVEC = 16 * 512 // _L


def _mesh():
    return plsc.VectorSubcoreMesh(
        core_axis_name="c", subcore_axis_name="s", num_cores=_NC, num_subcores=_NS
    )


@functools.lru_cache(maxsize=None)
def _build_stats_sc():
    return pl.kernel(
        _stats_body,
        mesh=_mesh(),
        out_type=jax.ShapeDtypeStruct((_NW, _NSTAT * _L), jnp.float32),
        compiler_params=pltpu.CompilerParams(use_tc_tiling_on_sc=True),
        scratch_types=[
            pltpu.VMEM((2, 8, 512), jnp.float32),
            pltpu.VMEM((2, 8, 512), jnp.int32),
            pltpu.VMEM((_NSTAT * _L,), jnp.float32),
            pltpu.SemaphoreType.DMA,
            pltpu.SemaphoreType.DMA,
        ],
    )


def _stats_body(loss_hbm, so_hbm, out_hbm, lbuf, obuf, stage, sem0, sem1):
    wid = lax.axis_index("s") * _NC + lax.axis_index("c")
    zeros = jnp.zeros((_L,), jnp.float32)
    sems = (sem0, sem1)

    def chunk_slices(u):
        band = wid * _BPW_SC + u
        b = band // 64
        r0 = (band % 64) * 8
        return b, r0

    def start(u, slot):
        b, r0 = chunk_slices(u)
        pltpu.async_copy(loss_hbm.at[b, pl.ds(r0, 8), :], lbuf.at[slot], sems[slot])
        pltpu.async_copy(so_hbm.at[b, pl.ds(r0, 8), :], obuf.at[slot], sems[slot])

    def wait(u, slot):
        b, r0 = chunk_slices(u)
        pltpu.make_async_copy(loss_hbm.at[b, pl.ds(r0, 8), :], lbuf.at[slot], sems[slot]).wait()
        pltpu.make_async_copy(so_hbm.at[b, pl.ds(r0, 8), :], obuf.at[slot], sems[slot]).wait()

    def compute(slot, accs):
        def vec_body(i, a):
            ns_cnt, h_cnt, h_sum, hs_cnt, hs_sum = a
            r = i // 32
            c = (i % 32) * _L
            ls = lbuf[slot, r, pl.ds(c, _L)]
            so = obuf[slot, r, pl.ds(c, _L)]
            one = jnp.ones((_L,), jnp.float32)
            zero = jnp.zeros((_L,), jnp.float32)
            so_f = so.astype(jnp.float32)
            hard = ls > _THR
            hard_f = jnp.where(hard, one, zero)
            hs_f = hard_f * so_f
            return (
                ns_cnt + so_f,
                h_cnt + hard_f,
                h_sum + jnp.where(hard, ls, zero),
                hs_cnt + hs_f,
                hs_sum + hs_f * ls,
            )

        return lax.fori_loop(0, _BVEC, vec_body, accs, unroll=8)

    start(0, 0)

    def loop_body(jj, accs):
        u0 = 2 * jj
        start(u0 + 1, 1)
        wait(u0, 0)
        accs = compute(0, accs)

        @pl.when(jj + 1 < _BPW_SC // 2)
        def _():
            start(u0 + 2, 0)

        wait(u0 + 1, 1)
        return compute(1, accs)

    accs = lax.fori_loop(0, _BPW_SC // 2, loop_body, (zeros,) * _NSTAT)
    for i in range(_NSTAT):
        stage[pl.ds(i * _L, _L)] = accs[i]
    pltpu.sync_copy(stage, out_hbm.at[wid])


@functools.lru_cache(maxsize=None)
def _build_valid_tc():
    return pl.pallas_call(
        _valid_tc_body,
        out_shape=jax.ShapeDtypeStruct((1, 1), jnp.float32),
        out_specs=pl.BlockSpec(memory_space=pltpu.SMEM),
    )


def _valid_tc_body(seg_ref, out_ref):
    valid = seg_ref[...] != _IGNORE
    out_ref[0, 0] = jnp.sum(jnp.where(valid, jnp.float32(1.0), jnp.float32(0.0)))


@functools.lru_cache(maxsize=None)
def _build_stats_tc():
    return pl.pallas_call(
        _stats_tc_body,
        grid=(_TC_BATCHES,),
        in_specs=[
            pl.BlockSpec((1, 512, 512), lambda i: (_SC_BATCHES + i, 0, 0)),
            pl.BlockSpec((1, 512, 512), lambda i: (_SC_BATCHES + i, 0, 0)),
        ],
        out_shape=tuple(
            jax.ShapeDtypeStruct((1, 1), jnp.float32) for _ in range(_NSTAT)
        ),
        out_specs=tuple(
            pl.BlockSpec(memory_space=pltpu.SMEM) for _ in range(_NSTAT)
        ),
    )


def _stats_tc_body(loss_ref, so_ref, ns_o, hc_o, hsum_o, hsc_o, hss_o):
    @pl.when(pl.program_id(0) == 0)
    def _():
        for o in (ns_o, hc_o, hsum_o, hsc_o, hss_o):
            o[0, 0] = jnp.float32(0.0)

    ls = loss_ref[0]
    so_f = so_ref[0].astype(jnp.float32)
    hard = ls > _THR
    hard_f = jnp.where(hard, jnp.float32(1.0), jnp.float32(0.0))
    hs_f = hard_f * so_f
    ns_o[0, 0] += jnp.sum(so_f)
    hc_o[0, 0] += jnp.sum(hard_f)
    hsum_o[0, 0] += jnp.sum(jnp.where(hard, ls, jnp.float32(0.0)))
    hsc_o[0, 0] += jnp.sum(hs_f)
    hss_o[0, 0] += jnp.sum(hs_f * ls)


@functools.lru_cache(maxsize=None)
def _build_hist_sc():
    return pl.kernel(
        _hist_body,
        mesh=_mesh(),
        out_type=(
            jax.ShapeDtypeStruct((_NW, 2 * _B), jnp.float32),
            jax.ShapeDtypeStruct((_NW, 2 * _B), jnp.float32),
        ),
        compiler_params=pltpu.CompilerParams(
            needs_layout_passes=False, use_tc_tiling_on_sc=True
        ),
        scratch_types=[
            pltpu.VMEM((8, 512), jnp.float32),
            pltpu.VMEM((8, 512), jnp.int32),
            pltpu.VMEM((2 * _B,), jnp.float32),
            pltpu.VMEM((2 * _B,), jnp.float32),
        ],
    )


def _hist_body(loss_hbm, so_hbm, cnt_hbm, sum_hbm, lbuf, obuf, hcnt, hsum):
    wid = lax.axis_index("s") * _NC + lax.axis_index("c")
    zeros = jnp.zeros((_L,), jnp.float32)
    ones = jnp.ones((_L,), jnp.float32)

    def zero_body(i, c):
        hcnt[pl.ds(i * _L, _L)] = zeros
        hsum[pl.ds(i * _L, _L)] = zeros
        return c

    lax.fori_loop(0, 2 * _B // _L, zero_body, 0)

    def band_body(t, carry):
        band = wid * _BPW + t
        b = band // 64
        r0 = (band % 64) * 8
        pltpu.sync_copy(loss_hbm.at[b, pl.ds(r0, 8), :], lbuf)
        pltpu.sync_copy(so_hbm.at[b, pl.ds(r0, 8), :], obuf)

        def vec_body(i, cc):
            r = i // 32
            c = (i % 32) * _L
            ls = lbuf[r, pl.ds(c, _L)]
            so = obuf[r, pl.ds(c, _L)]
            g = jnp.clip(so, 0, 1)
            bn = jnp.clip((ls * jnp.float32(_B)).astype(jnp.int32), 0, _B - 1)
            idx = bn + g * _B
            plsc.addupdate_scatter(hcnt, [idx], ones)
            plsc.addupdate_scatter(hsum, [idx], ls)
            return cc

        return lax.fori_loop(0, _BVEC, vec_body, carry)

    lax.fori_loop(0, _BPW, band_body, 0)
    pltpu.sync_copy(hcnt, cnt_hbm.at[wid])
    pltpu.sync_copy(hsum, sum_hbm.at[wid])


def _topk_sum_from_hist(cnt, sm, k):
    c = cnt[::-1]
    s = sm[::-1]
    cum = jnp.cumsum(c)
    take = jnp.clip(k.astype(jnp.float32) - (cum - c), 0.0, c)
    return jnp.sum(take * (s / jnp.maximum(c, 1.0)))


def kernel(loss, labels_seg, labels_so):
    parts = _build_stats_sc()(loss, labels_so)
    n_valid = _build_valid_tc()(labels_seg)[0, 0]
    tc_parts = _build_stats_tc()(loss, labels_so)
    p = jnp.sum(parts.reshape(_NW, _NSTAT, _L), axis=(0, 2))
    numel_s, h_cnt, h_sum, hs_cnt, hs_sum = (
        p[i] + tc_parts[i][0, 0] for i in range(_NSTAT)
    )

    n_min = (n_valid * jnp.float32(_RATIO)).astype(jnp.int32)
    cnt_s = hs_cnt.astype(jnp.int32)
    cnt_l = (h_cnt - hs_cnt).astype(jnp.int32)
    numel_s_i = numel_s.astype(jnp.int32)
    numel_l_i = jnp.int32(_N) - numel_s_i
    sum_s = hs_sum
    sum_l = h_sum - hs_sum

    k_s = jnp.where(cnt_s >= n_min, cnt_s, jnp.minimum(numel_s_i, n_min))
    k_l = jnp.where(cnt_l >= n_min, cnt_l, jnp.minimum(numel_l_i, n_min))
    need_s = cnt_s < n_min
    need_l = cnt_l < n_min

    def _fallback(_):
        hcnt, hsum = _build_hist_sc()(loss, labels_so)
        hc = jnp.sum(hcnt.reshape(_NW, 2, _B), axis=0)
        hs = jnp.sum(hsum.reshape(_NW, 2, _B), axis=0)
        return (
            _topk_sum_from_hist(hc[1], hs[1], k_s),
            _topk_sum_from_hist(hc[0], hs[0], k_l),
        )

    def _easy(_):
        return sum_s, sum_l

    fb_s, fb_l = lax.cond(need_s | need_l, _fallback, _easy, None)
    s_s = jnp.where(need_s, fb_s, sum_s)
    s_l = jnp.where(need_l, fb_l, sum_l)
    return (s_s + s_l) / (k_s + k_l).astype(jnp.float32)


# speedup vs baseline: 1.0993x; 1.0519x over previous
import functools

import jax
import jax.numpy as jnp
from jax import lax
from jax.experimental import pallas as pl
from jax.experimental.pallas import tpu as pltpu
from jax.experimental.pallas import tpu_sc as plsc

_IGNORE = 255
_RATIO = 0.1
_THR = 0.5
_SHAPE = (16, 512, 512)
_N = 16 * 512 * 512

_NC = 2
_NS = 16
_L = 16
_NW = _NC * _NS
_NBAND = 1024
_BPW = _NBAND // _NW
_BVEC = 8 * 512 // _L
_NSTAT = 4
_B = 1024

_SC_BATCHES = 10
_TC_BATCHES = 16 - _SC_BATCHES
_BPW_SC = _SC_BATCHES * 64 // _NW
_CPW = _BPW_SC // 2
_CVEC = 16 * 512 // _L


def _mesh():
    return plsc.VectorSubcoreMesh(
        core_axis_name="c", subcore_axis_name="s", num_cores=_NC, num_subcores=_NS
    )


@functools.lru_cache(maxsize=None)
def _build_stats_sc():
    return pl.kernel(
        _stats_body,
        mesh=_mesh(),
        out_type=jax.ShapeDtypeStruct((_NW, _NSTAT * _L), jnp.float32),
        compiler_params=pltpu.CompilerParams(use_tc_tiling_on_sc=True),
        scratch_types=[
            pltpu.VMEM((2, 8, 512), jnp.float32),
            pltpu.VMEM((2, 8, 512), jnp.int32),
            pltpu.VMEM((_NSTAT * _L,), jnp.float32),
            pltpu.SemaphoreType.DMA,
            pltpu.SemaphoreType.DMA,
        ],
    )


def _stats_body(loss_hbm, so_hbm, out_hbm, lbuf, obuf, stage, sem0, sem1):
    wid = lax.axis_index("s") * _NC + lax.axis_index("c")
    zeros = jnp.zeros((_L,), jnp.float32)
    sems = (sem0, sem1)

    def chunk_slices(u):
        band = wid * _BPW_SC + u
        b = band // 64
        r0 = (band % 64) * 8
        return b, r0

    def start(u, slot):
        b, r0 = chunk_slices(u)
        pltpu.async_copy(loss_hbm.at[b, pl.ds(r0, 8), :], lbuf.at[slot], sems[slot])
        pltpu.async_copy(so_hbm.at[b, pl.ds(r0, 8), :], obuf.at[slot], sems[slot])

    def wait(u, slot):
        b, r0 = chunk_slices(u)
        pltpu.make_async_copy(loss_hbm.at[b, pl.ds(r0, 8), :], lbuf.at[slot], sems[slot]).wait()
        pltpu.make_async_copy(so_hbm.at[b, pl.ds(r0, 8), :], obuf.at[slot], sems[slot]).wait()

    def compute(slot, accs):
        def vec_body(i, a):
            h_cnt, h_sum, hs_cnt, hs_sum = a
            r = i // 32
            c = (i % 32) * _L
            ls = lbuf[slot, r, pl.ds(c, _L)]
            so = obuf[slot, r, pl.ds(c, _L)]
            one = jnp.ones((_L,), jnp.float32)
            zero = jnp.zeros((_L,), jnp.float32)
            so_f = so.astype(jnp.float32)
            hard = ls > _THR
            hard_f = jnp.where(hard, one, zero)
            hs_f = hard_f * so_f
            return (
                h_cnt + hard_f,
                h_sum + jnp.where(hard, ls, zero),
                hs_cnt + hs_f,
                hs_sum + hs_f * ls,
            )

        return lax.fori_loop(0, _BVEC, vec_body, accs, unroll=8)

    start(0, 0)

    def loop_body(jj, accs):
        u0 = 2 * jj
        start(u0 + 1, 1)
        wait(u0, 0)
        accs = compute(0, accs)

        @pl.when(jj + 1 < _BPW_SC // 2)
        def _():
            start(u0 + 2, 0)

        wait(u0 + 1, 1)
        return compute(1, accs)

    accs = lax.fori_loop(0, _BPW_SC // 2, loop_body, (zeros,) * _NSTAT)
    for i in range(_NSTAT):
        stage[pl.ds(i * _L, _L)] = accs[i]
    pltpu.sync_copy(stage, out_hbm.at[wid])


@functools.lru_cache(maxsize=None)
def _build_valid_tc():
    return pl.pallas_call(
        _valid_tc_body,
        out_shape=jax.ShapeDtypeStruct((1, 1), jnp.float32),
        out_specs=pl.BlockSpec(memory_space=pltpu.SMEM),
    )


def _valid_tc_body(seg_ref, out_ref):
    valid = seg_ref[...] != _IGNORE
    out_ref[0, 0] = jnp.sum(jnp.where(valid, jnp.float32(1.0), jnp.float32(0.0)))


@functools.lru_cache(maxsize=None)
def _build_stats_tc():
    return pl.pallas_call(
        _stats_tc_body,
        grid=(_TC_BATCHES,),
        in_specs=[
            pl.BlockSpec((1, 512, 512), lambda i: (_SC_BATCHES + i, 0, 0)),
            pl.BlockSpec((1, 512, 512), lambda i: (_SC_BATCHES + i, 0, 0)),
        ],
        out_shape=tuple(
            jax.ShapeDtypeStruct((1, 1), jnp.float32) for _ in range(_NSTAT)
        ),
        out_specs=tuple(
            pl.BlockSpec(memory_space=pltpu.SMEM) for _ in range(_NSTAT)
        ),
    )


def _stats_tc_body(loss_ref, so_ref, hc_o, hsum_o, hsc_o, hss_o):
    @pl.when(pl.program_id(0) == 0)
    def _():
        for o in (hc_o, hsum_o, hsc_o, hss_o):
            o[0, 0] = jnp.float32(0.0)

    ls = loss_ref[0]
    so_f = so_ref[0].astype(jnp.float32)
    hard = ls > _THR
    hard_f = jnp.where(hard, jnp.float32(1.0), jnp.float32(0.0))
    hs_f = hard_f * so_f
    hc_o[0, 0] += jnp.sum(hard_f)
    hsum_o[0, 0] += jnp.sum(jnp.where(hard, ls, jnp.float32(0.0)))
    hsc_o[0, 0] += jnp.sum(hs_f)
    hss_o[0, 0] += jnp.sum(hs_f * ls)


@functools.lru_cache(maxsize=None)
def _build_hist_sc():
    return pl.kernel(
        _hist_body,
        mesh=_mesh(),
        out_type=(
            jax.ShapeDtypeStruct((_NW, 2 * _B), jnp.float32),
            jax.ShapeDtypeStruct((_NW, 2 * _B), jnp.float32),
        ),
        compiler_params=pltpu.CompilerParams(
            needs_layout_passes=False, use_tc_tiling_on_sc=True
        ),
        scratch_types=[
            pltpu.VMEM((8, 512), jnp.float32),
            pltpu.VMEM((8, 512), jnp.int32),
            pltpu.VMEM((2 * _B,), jnp.float32),
            pltpu.VMEM((2 * _B,), jnp.float32),
        ],
    )


def _hist_body(loss_hbm, so_hbm, cnt_hbm, sum_hbm, lbuf, obuf, hcnt, hsum):
    wid = lax.axis_index("s") * _NC + lax.axis_index("c")
    zeros = jnp.zeros((_L,), jnp.float32)
    ones = jnp.ones((_L,), jnp.float32)

    def zero_body(i, c):
        hcnt[pl.ds(i * _L, _L)] = zeros
        hsum[pl.ds(i * _L, _L)] = zeros
        return c

    lax.fori_loop(0, 2 * _B // _L, zero_body, 0)

    def band_body(t, carry):
        band = wid * _BPW + t
        b = band // 64
        r0 = (band % 64) * 8
        pltpu.sync_copy(loss_hbm.at[b, pl.ds(r0, 8), :], lbuf)
        pltpu.sync_copy(so_hbm.at[b, pl.ds(r0, 8), :], obuf)

        def vec_body(i, cc):
            r = i // 32
            c = (i % 32) * _L
            ls = lbuf[r, pl.ds(c, _L)]
            so = obuf[r, pl.ds(c, _L)]
            g = jnp.clip(so, 0, 1)
            bn = jnp.clip((ls * jnp.float32(_B)).astype(jnp.int32), 0, _B - 1)
            idx = bn + g * _B
            plsc.addupdate_scatter(hcnt, [idx], ones)
            plsc.addupdate_scatter(hsum, [idx], ls)
            return cc

        return lax.fori_loop(0, _BVEC, vec_body, carry)

    lax.fori_loop(0, _BPW, band_body, 0)
    pltpu.sync_copy(hcnt, cnt_hbm.at[wid])
    pltpu.sync_copy(hsum, sum_hbm.at[wid])


def _topk_sum_from_hist(cnt, sm, k):
    c = cnt[::-1]
    s = sm[::-1]
    cum = jnp.cumsum(c)
    take = jnp.clip(k.astype(jnp.float32) - (cum - c), 0.0, c)
    return jnp.sum(take * (s / jnp.maximum(c, 1.0)))


def kernel(loss, labels_seg, labels_so):
    parts = _build_stats_sc()(loss, labels_so)
    n_valid = _build_valid_tc()(labels_seg)[0, 0]
    tc_parts = _build_stats_tc()(loss, labels_so)
    p = jnp.sum(parts.reshape(_NW, _NSTAT, _L), axis=(0, 2))
    h_cnt, h_sum, hs_cnt, hs_sum = (
        p[i] + tc_parts[i][0, 0] for i in range(_NSTAT)
    )

    n_min = (n_valid * jnp.float32(_RATIO)).astype(jnp.int32)
    cnt_s = hs_cnt.astype(jnp.int32)
    cnt_l = (h_cnt - hs_cnt).astype(jnp.int32)
    sum_s = hs_sum
    sum_l = h_sum - hs_sum
    need_s = cnt_s < n_min
    need_l = cnt_l < n_min

    def _fallback(_):
        hcnt, hsum = _build_hist_sc()(loss, labels_so)
        hc = jnp.sum(hcnt.reshape(_NW, 2, _B), axis=0)
        hs = jnp.sum(hsum.reshape(_NW, 2, _B), axis=0)
        numel_s_i = jnp.sum(hc[1]).astype(jnp.int32)
        numel_l_i = jnp.int32(_N) - numel_s_i
        k_s = jnp.where(need_s, jnp.minimum(numel_s_i, n_min), cnt_s)
        k_l = jnp.where(need_l, jnp.minimum(numel_l_i, n_min), cnt_l)
        s_s = jnp.where(need_s, _topk_sum_from_hist(hc[1], hs[1], k_s), sum_s)
        s_l = jnp.where(need_l, _topk_sum_from_hist(hc[0], hs[0], k_l), sum_l)
        return s_s, s_l, k_s, k_l

    def _easy(_):
        return sum_s, sum_l, cnt_s, cnt_l

    s_s, s_l, k_s, k_l = lax.cond(need_s | need_l, _fallback, _easy, None)
    return (s_s + s_l) / (k_s + k_l).astype(jnp.float32)
